# Initial kernel scaffold; baseline (speedup 1.0000x reference)
import jax
import jax.numpy as jnp
from jax.experimental import pallas as pl


def kernel(apig_edge_index, apig_feat, fcg_edge_index, fcg_feat, W_a1, b_a1, W_a2, b_a2, W_f1, b_f1, W_f2, b_f2, attn_W, attn_b, cls_W, cls_b):
    raise NotImplementedError("write your pallas kernel here")


# speedup vs baseline: 2.8708x; 2.8708x over previous
import functools

import jax
import jax.numpy as jnp
from jax import lax
from jax.experimental import pallas as pl
from jax.experimental.pallas import tpu as pltpu
from jax.experimental.pallas import tpu_sc as plsc

N = 10000
E = 320000
D = 128
C = 10

NC = 2
NS = 16
RPT = 640
N1 = NS * RPT
K = 128
NCH = 157
EPT = NCH * K
E_PAD = NS * EPT
RB = 128
NBLK = N1 // RB

_MESH = plsc.VectorSubcoreMesh(core_axis_name="c", subcore_axis_name="s")


@functools.partial(
    pl.kernel,
    out_type=jax.ShapeDtypeStruct((4 * N1, D), jnp.float32),
    mesh=_MESH,
    scratch_types=[
        pltpu.VMEM_SHARED((N1, D), jnp.float32),
        pltpu.VMEM((K, D), jnp.float32),
        pltpu.VMEM((K,), jnp.int32),
        pltpu.SemaphoreType.DMA,
    ],
)
def _sc_degrees(e_ref, ones_ref, zeros_ref, deg_ref, hist, rows, idxb, sem):
    c = lax.axis_index("c")
    s = lax.axis_index("s")
    base = s * RPT
    ebase = s * EPT
    g_e = c * 3 * E_PAD

    for dirn in (0, 1):
        pltpu.sync_copy(zeros_ref, rows)
        for k in range(RPT // K):
            pltpu.sync_copy(rows, hist.at[pl.ds(base + k * K, K)])
        plsc.subcore_barrier()
        pltpu.sync_copy(ones_ref, rows)

        def hist_body(i, _):
            off = g_e + ebase + i * K + dirn * E_PAD
            pltpu.sync_copy(e_ref.at[pl.ds(off, K)], idxb)
            pltpu.sync_copy(rows, hist.at[idxb], add=True)
            return 0

        lax.fori_loop(0, NCH, hist_body, 0)
        plsc.subcore_barrier()

        for k in range(RPT // K):
            r0 = base + k * K
            pltpu.sync_copy(hist.at[pl.ds(r0, K)], rows)
            pltpu.sync_copy(rows, deg_ref.at[pl.ds((c * 2 + dirn) * N1 + r0, K)])


@functools.partial(
    pl.kernel,
    out_type=jax.ShapeDtypeStruct((2 * N1, D), jnp.float32),
    mesh=_MESH,
    scratch_types=[
        pltpu.VMEM_SHARED((N1, D), jnp.float32),
        pltpu.VMEM((K, D), jnp.float32),
        pltpu.VMEM((K, D), jnp.float32),
        pltpu.VMEM((K,), jnp.int32),
        pltpu.VMEM((K,), jnp.int32),
        pltpu.SemaphoreType.DMA,
    ],
)
def _sc_conv(e_ref, h_ref, zeros_ref, agg_ref, acc, zrows, rows, isrc, idst,
             sem):
    c = lax.axis_index("c")
    s = lax.axis_index("s")
    base = s * RPT
    ebase = s * EPT
    g_e = c * 3 * E_PAD

    pltpu.sync_copy(zeros_ref, zrows)
    for k in range(RPT // K):
        pltpu.sync_copy(zrows, acc.at[pl.ds(base + k * K, K)])
    plsc.subcore_barrier()

    def conv_body(i, _):
        off = g_e + ebase + i * K
        pltpu.sync_copy(e_ref.at[pl.ds(off + 2 * E_PAD, K)], isrc)
        pltpu.sync_copy(e_ref.at[pl.ds(off + E_PAD, K)], idst)
        pltpu.async_copy(h_ref.at[isrc], rows, sem).wait()
        pltpu.async_copy(rows, acc.at[idst], sem, add=True).wait()
        return 0

    lax.fori_loop(0, NCH, conv_body, 0)
    plsc.subcore_barrier()

    for k in range(RPT // K):
        r0 = base + k * K
        pltpu.sync_copy(acc.at[pl.ds(r0, K)], rows)
        pltpu.sync_copy(rows, agg_ref.at[pl.ds(c * N1 + r0, K)])


def _tc_norm_body(deg_s_ref, deg_d_ref, x_ref, h_ref, ns_ref, nd_ref):
    ns = lax.rsqrt(jnp.maximum(deg_s_ref[0][:, 0:1], 1.0))
    nd = lax.rsqrt(jnp.maximum(deg_d_ref[0][:, 0:1], 1.0))
    ns_ref[0] = jnp.broadcast_to(ns, (RB, 16))
    nd_ref[0] = jnp.broadcast_to(nd, (RB, 16))
    h_ref[0] = x_ref[0] * ns


def _tc_norms_and_h1(deg, x):
    h1, ns, nd = pl.pallas_call(
        _tc_norm_body,
        grid=(2, NBLK),
        in_specs=[
            pl.BlockSpec((1, RB, D), lambda g, i: (2 * g, i, 0)),
            pl.BlockSpec((1, RB, D), lambda g, i: (2 * g + 1, i, 0)),
            pl.BlockSpec((1, RB, D), lambda g, i: (g, i, 0)),
        ],
        out_specs=[
            pl.BlockSpec((1, RB, D), lambda g, i: (g, i, 0)),
            pl.BlockSpec((1, RB, 16), lambda g, i: (g, i, 0)),
            pl.BlockSpec((1, RB, 16), lambda g, i: (g, i, 0)),
        ],
        out_shape=[
            jax.ShapeDtypeStruct((2, N1, D), jnp.float32),
            jax.ShapeDtypeStruct((2, N1, 16), jnp.float32),
            jax.ShapeDtypeStruct((2, N1, 16), jnp.float32),
        ],
    )(deg, deg, x)
    return h1, ns, nd


def _tc_mm_body(a_ref, w_ref, b_ref, nd_ref, ns_ref, o_ref):
    nd = nd_ref[0][:, 0:1]
    a = a_ref[0] * nd
    acc = jnp.dot(a, w_ref[0], preferred_element_type=jnp.float32)
    r = jnp.maximum(acc + b_ref[0], 0.0)
    o_ref[0] = r * ns_ref[0][:, 0:1]


def _tc_matmul_relu(agg, W, b, nd, ns):
    return pl.pallas_call(
        _tc_mm_body,
        grid=(2, NBLK),
        in_specs=[
            pl.BlockSpec((1, RB, D), lambda g, i: (g, i, 0)),
            pl.BlockSpec((1, D, D), lambda g, i: (g, 0, 0)),
            pl.BlockSpec((1, 1, D), lambda g, i: (g, 0, 0)),
            pl.BlockSpec((1, RB, 16), lambda g, i: (g, i, 0)),
            pl.BlockSpec((1, RB, 16), lambda g, i: (g, i, 0)),
        ],
        out_specs=pl.BlockSpec((1, RB, D), lambda g, i: (g, i, 0)),
        out_shape=jax.ShapeDtypeStruct((2, N1, D), jnp.float32),
    )(agg, W, b.reshape(2, 1, D), nd, ns)


def _tc_pool_body(a_ref, w_ref, b_ref, nd_ref, o_ref, acc_ref):
    g = pl.program_id(0)
    i = pl.program_id(1)
    a = a_ref[0] * nd_ref[0][:, 0:1]
    x = jnp.dot(a, w_ref[0], preferred_element_type=jnp.float32)
    x = jnp.maximum(x + b_ref[0], 0.0)
    rowid = i * RB + lax.broadcasted_iota(jnp.int32, (RB, 1), 0)
    x = jnp.where(rowid < N, x, 0.0)
    ssum = jnp.sum(x, axis=0, keepdims=True)
    smax = jnp.max(x, axis=0, keepdims=True)
    red = jnp.where(g == 0, ssum, smax)

    @pl.when(i == 0)
    def _():
        acc_ref[...] = red

    @pl.when(i > 0)
    def _():
        a0 = acc_ref[...]
        acc_ref[...] = jnp.where(g == 0, a0 + red, jnp.maximum(a0, red))

    @pl.when(i == NBLK - 1)
    def _():
        r = acc_ref[...]
        o_ref[0] = jnp.where(g == 0, r * jnp.float32(1.0 / N), r)


def _tc_matmul_pool(agg, W, b, nd):
    return pl.pallas_call(
        _tc_pool_body,
        grid=(2, NBLK),
        in_specs=[
            pl.BlockSpec((1, RB, D), lambda g, i: (g, i, 0)),
            pl.BlockSpec((1, D, D), lambda g, i: (g, 0, 0)),
            pl.BlockSpec((1, 1, D), lambda g, i: (g, 0, 0)),
            pl.BlockSpec((1, RB, 16), lambda g, i: (g, i, 0)),
        ],
        out_specs=pl.BlockSpec((1, 1, D), lambda g, i: (g, 0, 0)),
        out_shape=jax.ShapeDtypeStruct((2, 1, D), jnp.float32),
        scratch_shapes=[pltpu.VMEM((1, D), jnp.float32)],
    )(agg, W, b.reshape(2, 1, D), nd)


def _tc_head_body(p_ref, w_ref, b_ref, o_ref):
    p = p_ref[...]
    mn = jnp.min(p, axis=1, keepdims=True)
    mx = jnp.max(p, axis=1, keepdims=True)
    q = (p - mn) / (mx - mn)
    emb = q[0:1, :] + q[1:2, :]
    o_ref[...] = (
        jnp.dot(emb, w_ref[...], preferred_element_type=jnp.float32) + b_ref[...]
    )


def _tc_head(pooled, cls_W, cls_b):
    return pl.pallas_call(
        _tc_head_body,
        out_shape=jax.ShapeDtypeStruct((1, C), jnp.float32),
    )(pooled.reshape(2, D), cls_W, cls_b.reshape(1, C))


def kernel(apig_edge_index, apig_feat, fcg_edge_index, fcg_feat,
           W_a1, b_a1, W_a2, b_a2, W_f1, b_f1, W_f2, b_f2,
           attn_W, attn_b, cls_W, cls_b):
    f32 = jnp.float32
    xpad = jnp.zeros((N1 - N, D), f32)
    x = jnp.stack([jnp.concatenate([apig_feat.astype(f32), xpad], axis=0),
                   jnp.concatenate([fcg_feat.astype(f32), xpad], axis=0)])

    epad = jnp.full((E_PAD - E,), N, jnp.int32)

    def prep_edges(ei, g):
        srcp = jnp.concatenate([ei[0].astype(jnp.int32), epad])
        dstp = jnp.concatenate([ei[1].astype(jnp.int32), epad])
        return jnp.concatenate([srcp, dstp, srcp + jnp.int32(g * N1)])

    e_flat = jnp.concatenate(
        [prep_edges(apig_edge_index, 0), prep_edges(fcg_edge_index, 1)])

    onesD = jnp.ones((K, D), f32)
    zerosD = jnp.zeros((K, D), f32)

    deg = _sc_degrees(e_flat, onesD, zerosD)
    h1, ns, nd = _tc_norms_and_h1(deg.reshape(4, N1, D), x)

    agg1 = _sc_conv(e_flat, h1.reshape(2 * N1, D), zerosD)

    W1 = jnp.stack([W_a1, W_f1])
    b1 = jnp.stack([b_a1, b_f1])
    h2 = _tc_matmul_relu(agg1.reshape(2, N1, D), W1, b1, nd, ns)

    agg2 = _sc_conv(e_flat, h2.reshape(2 * N1, D), zerosD)

    W2 = jnp.stack([W_a2, W_f2])
    b2 = jnp.stack([b_a2, b_f2])
    pooled = _tc_matmul_pool(agg2.reshape(2, N1, D), W2, b2, nd)

    out = _tc_head(pooled, cls_W, cls_b)
    return out.reshape(C)


# speedup vs baseline: 4.3630x; 1.5198x over previous
import functools

import jax
import jax.numpy as jnp
from jax import lax
from jax.experimental import pallas as pl
from jax.experimental.pallas import tpu as pltpu
from jax.experimental.pallas import tpu_sc as plsc

N = 10000
E = 320000
D = 128
C = 10

NC = 2
NS = 16
RPT = 640
N1 = NS * RPT
K = 128
NCH = 160
EPT = NCH * K
E_PAD = NS * EPT
ECHUNKS = E_PAD // K
NBI = 8
NEB = NCH // NBI
RB = 128
NBLK = N1 // RB

_MESH = plsc.VectorSubcoreMesh(core_axis_name="c", subcore_axis_name="s")


@functools.partial(
    pl.kernel,
    out_type=jax.ShapeDtypeStruct((4 * N1, D), jnp.float32),
    mesh=_MESH,
    scratch_types=[
        pltpu.VMEM_SHARED((N1, D), jnp.float32),
        pltpu.VMEM((K, D), jnp.float32),
        pltpu.VMEM((NBI, K), jnp.int32),
        pltpu.SemaphoreType.DMA,
    ],
)
def _sc_degrees(e2_ref, ones_ref, zeros_ref, deg_ref, hist, rows, idxb, sem):
    c = lax.axis_index("c")
    s = lax.axis_index("s")
    base = s * RPT

    for dirn in (0, 1):
        pltpu.sync_copy(zeros_ref, rows)
        for k in range(RPT // K):
            pltpu.sync_copy(rows, hist.at[pl.ds(base + k * K, K)])
        plsc.subcore_barrier()
        pltpu.sync_copy(ones_ref, rows)
        row0 = (c * 3 + dirn) * ECHUNKS + s * NCH

        def blk_body(b, _):
            pltpu.sync_copy(e2_ref.at[pl.ds(row0 + b * NBI, NBI)], idxb)
            for j in range(NBI):
                pltpu.async_copy(rows, hist.at[idxb.at[j]], sem, add=True)
            for j in range(NBI):
                pltpu.make_async_copy(rows, hist.at[idxb.at[j]], sem).wait()
            return 0

        lax.fori_loop(0, NEB, blk_body, 0)
        plsc.subcore_barrier()

        for k in range(RPT // K):
            r0 = base + k * K
            pltpu.sync_copy(hist.at[pl.ds(r0, K)], rows)
            pltpu.sync_copy(rows, deg_ref.at[pl.ds((c * 2 + dirn) * N1 + r0, K)])


@functools.partial(
    pl.kernel,
    out_type=jax.ShapeDtypeStruct((2 * N1, D), jnp.float32),
    mesh=_MESH,
    scratch_types=[
        pltpu.VMEM_SHARED((N1, D), jnp.float32),
        pltpu.VMEM((K, D), jnp.float32),
        pltpu.VMEM((K, D), jnp.float32),
        pltpu.VMEM((NBI, K), jnp.int32),
        pltpu.VMEM((NBI, K), jnp.int32),
        pltpu.VMEM((NBI, K), jnp.int32),
        pltpu.VMEM((NBI, K), jnp.int32),
        pltpu.VMEM((K,), jnp.int32),
        pltpu.SemaphoreType.DMA,
        pltpu.SemaphoreType.DMA,
        pltpu.SemaphoreType.DMA,
        pltpu.SemaphoreType.DMA,
    ],
)
def _sc_conv(e2_ref, h_ref, zeros_ref, sidx_ref, agg_ref,
             acc, rows0, rows1, isb0, idb0, isb1, idb1, sidx,
             gs0, gs1, ss0, ss1):
    c = lax.axis_index("c")
    s = lax.axis_index("s")
    base = s * RPT
    row_s = (c * 3 + 2) * ECHUNKS + s * NCH
    row_d = (c * 3 + 1) * ECHUNKS + s * NCH

    pltpu.sync_copy(zeros_ref, rows0)
    pltpu.sync_copy(zeros_ref, rows1)
    for k in range(RPT // K):
        pltpu.sync_copy(rows0, acc.at[pl.ds(base + k * K, K)])
    pltpu.sync_copy(sidx_ref, sidx)
    plsc.subcore_barrier()

    rows_l = (rows0, rows1)
    gs_l = (gs0, gs1)
    ss_l = (ss0, ss1)
    pltpu.async_copy(rows0, acc.at[sidx], ss0, add=True)
    pltpu.async_copy(rows1, acc.at[sidx], ss1, add=True)

    idx_l = ((isb0, idb0), (isb1, idb1))

    def blk_pair(t, _):
        for sb in range(2):
            b = t * 2 + sb
            isb, idb = idx_l[sb]
            pltpu.sync_copy(e2_ref.at[pl.ds(row_s + b * NBI, NBI)], isb)
            pltpu.sync_copy(e2_ref.at[pl.ds(row_d + b * NBI, NBI)], idb)
            for j in range(NBI):
                p = j & 1
                pltpu.make_async_copy(rows_l[p], acc.at[sidx], ss_l[p]).wait()
                pltpu.async_copy(h_ref.at[isb.at[j]], rows_l[p], gs_l[p])
                pltpu.make_async_copy(
                    h_ref.at[isb.at[j]], rows_l[p], gs_l[p]).wait()
                pltpu.async_copy(rows_l[p], acc.at[idb.at[j]], ss_l[p], add=True)
        return 0

    lax.fori_loop(0, NEB // 2, blk_pair, 0)
    pltpu.make_async_copy(rows0, acc.at[sidx], ss0).wait()
    pltpu.make_async_copy(rows1, acc.at[sidx], ss1).wait()
    plsc.subcore_barrier()

    for k in range(RPT // K):
        r0 = base + k * K
        pltpu.sync_copy(acc.at[pl.ds(r0, K)], rows0)
        pltpu.sync_copy(rows0, agg_ref.at[pl.ds(c * N1 + r0, K)])


def _tc_norm_body(deg_s_ref, deg_d_ref, x_ref, h_ref, ns_ref, nd_ref):
    ns = lax.rsqrt(jnp.maximum(deg_s_ref[0][:, 0:1], 1.0))
    nd = lax.rsqrt(jnp.maximum(deg_d_ref[0][:, 0:1], 1.0))
    ns_ref[0] = jnp.broadcast_to(ns, (RB, 16))
    nd_ref[0] = jnp.broadcast_to(nd, (RB, 16))
    h_ref[0] = x_ref[0] * ns


def _tc_norms_and_h1(deg, x):
    h1, ns, nd = pl.pallas_call(
        _tc_norm_body,
        grid=(2, NBLK),
        in_specs=[
            pl.BlockSpec((1, RB, D), lambda g, i: (2 * g, i, 0)),
            pl.BlockSpec((1, RB, D), lambda g, i: (2 * g + 1, i, 0)),
            pl.BlockSpec((1, RB, D), lambda g, i: (g, i, 0)),
        ],
        out_specs=[
            pl.BlockSpec((1, RB, D), lambda g, i: (g, i, 0)),
            pl.BlockSpec((1, RB, 16), lambda g, i: (g, i, 0)),
            pl.BlockSpec((1, RB, 16), lambda g, i: (g, i, 0)),
        ],
        out_shape=[
            jax.ShapeDtypeStruct((2, N1, D), jnp.float32),
            jax.ShapeDtypeStruct((2, N1, 16), jnp.float32),
            jax.ShapeDtypeStruct((2, N1, 16), jnp.float32),
        ],
    )(deg, deg, x)
    return h1, ns, nd


def _tc_mm_body(a_ref, w_ref, b_ref, nd_ref, ns_ref, o_ref):
    nd = nd_ref[0][:, 0:1]
    a = a_ref[0] * nd
    acc = jnp.dot(a, w_ref[0], preferred_element_type=jnp.float32)
    r = jnp.maximum(acc + b_ref[0], 0.0)
    o_ref[0] = r * ns_ref[0][:, 0:1]


def _tc_matmul_relu(agg, W, b, nd, ns):
    return pl.pallas_call(
        _tc_mm_body,
        grid=(2, NBLK),
        in_specs=[
            pl.BlockSpec((1, RB, D), lambda g, i: (g, i, 0)),
            pl.BlockSpec((1, D, D), lambda g, i: (g, 0, 0)),
            pl.BlockSpec((1, 1, D), lambda g, i: (g, 0, 0)),
            pl.BlockSpec((1, RB, 16), lambda g, i: (g, i, 0)),
            pl.BlockSpec((1, RB, 16), lambda g, i: (g, i, 0)),
        ],
        out_specs=pl.BlockSpec((1, RB, D), lambda g, i: (g, i, 0)),
        out_shape=jax.ShapeDtypeStruct((2, N1, D), jnp.float32),
    )(agg, W, b.reshape(2, 1, D), nd, ns)


def _tc_pool_body(a_ref, w_ref, b_ref, nd_ref, o_ref, acc_ref):
    g = pl.program_id(0)
    i = pl.program_id(1)
    a = a_ref[0] * nd_ref[0][:, 0:1]
    x = jnp.dot(a, w_ref[0], preferred_element_type=jnp.float32)
    x = jnp.maximum(x + b_ref[0], 0.0)
    rowid = i * RB + lax.broadcasted_iota(jnp.int32, (RB, 1), 0)
    x = jnp.where(rowid < N, x, 0.0)
    ssum = jnp.sum(x, axis=0, keepdims=True)
    smax = jnp.max(x, axis=0, keepdims=True)
    red = jnp.where(g == 0, ssum, smax)

    @pl.when(i == 0)
    def _():
        acc_ref[...] = red

    @pl.when(i > 0)
    def _():
        a0 = acc_ref[...]
        acc_ref[...] = jnp.where(g == 0, a0 + red, jnp.maximum(a0, red))

    @pl.when(i == NBLK - 1)
    def _():
        r = acc_ref[...]
        o_ref[0] = jnp.where(g == 0, r * jnp.float32(1.0 / N), r)


def _tc_matmul_pool(agg, W, b, nd):
    return pl.pallas_call(
        _tc_pool_body,
        grid=(2, NBLK),
        in_specs=[
            pl.BlockSpec((1, RB, D), lambda g, i: (g, i, 0)),
            pl.BlockSpec((1, D, D), lambda g, i: (g, 0, 0)),
            pl.BlockSpec((1, 1, D), lambda g, i: (g, 0, 0)),
            pl.BlockSpec((1, RB, 16), lambda g, i: (g, i, 0)),
        ],
        out_specs=pl.BlockSpec((1, 1, D), lambda g, i: (g, 0, 0)),
        out_shape=jax.ShapeDtypeStruct((2, 1, D), jnp.float32),
        scratch_shapes=[pltpu.VMEM((1, D), jnp.float32)],
    )(agg, W, b.reshape(2, 1, D), nd)


def _tc_head_body(p_ref, w_ref, b_ref, o_ref):
    p = p_ref[...]
    mn = jnp.min(p, axis=1, keepdims=True)
    mx = jnp.max(p, axis=1, keepdims=True)
    q = (p - mn) / (mx - mn)
    emb = q[0:1, :] + q[1:2, :]
    o_ref[...] = (
        jnp.dot(emb, w_ref[...], preferred_element_type=jnp.float32) + b_ref[...]
    )


def _tc_head(pooled, cls_W, cls_b):
    return pl.pallas_call(
        _tc_head_body,
        out_shape=jax.ShapeDtypeStruct((1, C), jnp.float32),
    )(pooled.reshape(2, D), cls_W, cls_b.reshape(1, C))


def kernel(apig_edge_index, apig_feat, fcg_edge_index, fcg_feat,
           W_a1, b_a1, W_a2, b_a2, W_f1, b_f1, W_f2, b_f2,
           attn_W, attn_b, cls_W, cls_b):
    f32 = jnp.float32
    xpad = jnp.zeros((N1 - N, D), f32)
    x = jnp.stack([jnp.concatenate([apig_feat.astype(f32), xpad], axis=0),
                   jnp.concatenate([fcg_feat.astype(f32), xpad], axis=0)])

    epad = (jnp.arange(E_PAD - E, dtype=jnp.int32) % (N1 - N)) + N

    def prep_edges(ei, g):
        srcp = jnp.concatenate([ei[0].astype(jnp.int32), epad])
        dstp = jnp.concatenate([ei[1].astype(jnp.int32), epad])
        return jnp.concatenate([srcp, dstp, srcp + jnp.int32(g * N1)])

    e2 = jnp.concatenate(
        [prep_edges(apig_edge_index, 0), prep_edges(fcg_edge_index, 1)]
    ).reshape(6 * ECHUNKS, K)

    onesD = jnp.ones((K, D), f32)
    zerosD = jnp.zeros((K, D), f32)
    sidx = (jnp.arange(K, dtype=jnp.int32) % (N1 - N)) + N

    deg = _sc_degrees(e2, onesD, zerosD)
    h1, ns, nd = _tc_norms_and_h1(deg.reshape(4, N1, D), x)

    agg1 = _sc_conv(e2, h1.reshape(2 * N1, D), zerosD, sidx)

    W1 = jnp.stack([W_a1, W_f1])
    b1 = jnp.stack([b_a1, b_f1])
    h2 = _tc_matmul_relu(agg1.reshape(2, N1, D), W1, b1, nd, ns)

    agg2 = _sc_conv(e2, h2.reshape(2 * N1, D), zerosD, sidx)

    W2 = jnp.stack([W_a2, W_f2])
    b2 = jnp.stack([b_a2, b_f2])
    pooled = _tc_matmul_pool(agg2.reshape(2, N1, D), W2, b2, nd)

    out = _tc_head(pooled, cls_W, cls_b)
    return out.reshape(C)


# speedup vs baseline: 5.4312x; 1.2448x over previous
import functools

import jax
import jax.numpy as jnp
from jax import lax
from jax.experimental import pallas as pl
from jax.experimental.pallas import tpu as pltpu
from jax.experimental.pallas import tpu_sc as plsc

N = 10000
E = 320000
D = 128
C = 10

NC = 2
NS = 16
RPT = 640
N1 = NS * RPT
K = 128
NCH = 160
EPT = NCH * K
E_PAD = NS * EPT
ECHUNKS = E_PAD // K
NBI = 8
NEB = NCH // NBI
RB = 2048
NBLK = N1 // RB

_MESH = plsc.VectorSubcoreMesh(core_axis_name="c", subcore_axis_name="s")


@functools.partial(
    pl.kernel,
    out_type=jax.ShapeDtypeStruct((4 * N1, D), jnp.float32),
    mesh=_MESH,
    scratch_types=[
        pltpu.VMEM_SHARED((N1, D), jnp.float32),
        pltpu.VMEM((K, D), jnp.float32),
        pltpu.VMEM((NBI, K), jnp.int32),
        pltpu.SemaphoreType.DMA,
    ],
)
def _sc_degrees(e2_ref, ones_ref, zeros_ref, deg_ref, hist, rows, idxb, sem):
    c = lax.axis_index("c")
    s = lax.axis_index("s")
    base = s * RPT

    for dirn in (0, 1):
        pltpu.sync_copy(zeros_ref, rows)
        for k in range(RPT // K):
            pltpu.sync_copy(rows, hist.at[pl.ds(base + k * K, K)])
        plsc.subcore_barrier()
        pltpu.sync_copy(ones_ref, rows)
        row0 = (c * 3 + dirn) * ECHUNKS + s * NCH

        def blk_body(b, _):
            pltpu.sync_copy(e2_ref.at[pl.ds(row0 + b * NBI, NBI)], idxb)
            for j in range(NBI):
                pltpu.async_copy(rows, hist.at[idxb.at[j]], sem, add=True)
            for j in range(NBI):
                pltpu.make_async_copy(rows, hist.at[idxb.at[j]], sem).wait()
            return 0

        lax.fori_loop(0, NEB, blk_body, 0)
        plsc.subcore_barrier()

        for k in range(RPT // K):
            r0 = base + k * K
            pltpu.sync_copy(hist.at[pl.ds(r0, K)], rows)
            pltpu.sync_copy(rows, deg_ref.at[pl.ds((c * 2 + dirn) * N1 + r0, K)])


@functools.partial(
    pl.kernel,
    out_type=jax.ShapeDtypeStruct((2 * N1, D), jnp.float32),
    mesh=_MESH,
    scratch_types=[
        pltpu.VMEM_SHARED((N1, D), jnp.float32),
        pltpu.VMEM((K, D), jnp.float32),
        pltpu.VMEM((K, D), jnp.float32),
        pltpu.VMEM((NBI, K), jnp.int32),
        pltpu.VMEM((NBI, K), jnp.int32),
        pltpu.VMEM((NBI, K), jnp.int32),
        pltpu.VMEM((NBI, K), jnp.int32),
        pltpu.VMEM((K,), jnp.int32),
        pltpu.SemaphoreType.DMA,
        pltpu.SemaphoreType.DMA,
        pltpu.SemaphoreType.DMA,
        pltpu.SemaphoreType.DMA,
    ],
)
def _sc_conv(e2_ref, h_ref, zeros_ref, sidx_ref, agg_ref,
             acc, rows0, rows1, isb0, idb0, isb1, idb1, sidx,
             gs0, gs1, ss0, ss1):
    c = lax.axis_index("c")
    s = lax.axis_index("s")
    base = s * RPT
    row_s = (c * 3 + 2) * ECHUNKS + s * NCH
    row_d = (c * 3 + 1) * ECHUNKS + s * NCH

    pltpu.sync_copy(zeros_ref, rows0)
    pltpu.sync_copy(zeros_ref, rows1)
    for k in range(RPT // K):
        pltpu.sync_copy(rows0, acc.at[pl.ds(base + k * K, K)])
    pltpu.sync_copy(sidx_ref, sidx)
    plsc.subcore_barrier()

    rows_l = (rows0, rows1)
    gs_l = (gs0, gs1)
    ss_l = (ss0, ss1)
    pltpu.async_copy(rows0, acc.at[sidx], ss0, add=True)
    pltpu.async_copy(rows1, acc.at[sidx], ss1, add=True)

    idx_l = ((isb0, idb0), (isb1, idb1))

    def blk_pair(t, _):
        for sb in range(2):
            b = t * 2 + sb
            isb, idb = idx_l[sb]
            pltpu.sync_copy(e2_ref.at[pl.ds(row_s + b * NBI, NBI)], isb)
            pltpu.sync_copy(e2_ref.at[pl.ds(row_d + b * NBI, NBI)], idb)
            for j in range(NBI):
                p = j & 1
                pltpu.make_async_copy(rows_l[p], acc.at[sidx], ss_l[p]).wait()
                pltpu.async_copy(h_ref.at[isb.at[j]], rows_l[p], gs_l[p])
                pltpu.make_async_copy(
                    h_ref.at[isb.at[j]], rows_l[p], gs_l[p]).wait()
                pltpu.async_copy(rows_l[p], acc.at[idb.at[j]], ss_l[p], add=True)
        return 0

    lax.fori_loop(0, NEB // 2, blk_pair, 0)
    pltpu.make_async_copy(rows0, acc.at[sidx], ss0).wait()
    pltpu.make_async_copy(rows1, acc.at[sidx], ss1).wait()
    plsc.subcore_barrier()

    for k in range(RPT // K):
        r0 = base + k * K
        pltpu.sync_copy(acc.at[pl.ds(r0, K)], rows0)
        pltpu.sync_copy(rows0, agg_ref.at[pl.ds(c * N1 + r0, K)])


def _tc_norm_body(deg_s_ref, deg_d_ref, x_ref, h_ref, ns_ref, nd_ref):
    ns = lax.rsqrt(jnp.maximum(deg_s_ref[0][:, 0:1], 1.0))
    nd = lax.rsqrt(jnp.maximum(deg_d_ref[0][:, 0:1], 1.0))
    ns_ref[0] = jnp.broadcast_to(ns, (RB, 16))
    nd_ref[0] = jnp.broadcast_to(nd, (RB, 16))
    h_ref[0] = x_ref[0] * ns


def _tc_norms_and_h1(deg, x):
    h1, ns, nd = pl.pallas_call(
        _tc_norm_body,
        grid=(2, NBLK),
        in_specs=[
            pl.BlockSpec((1, RB, D), lambda g, i: (2 * g, i, 0)),
            pl.BlockSpec((1, RB, D), lambda g, i: (2 * g + 1, i, 0)),
            pl.BlockSpec((1, RB, D), lambda g, i: (g, i, 0)),
        ],
        out_specs=[
            pl.BlockSpec((1, RB, D), lambda g, i: (g, i, 0)),
            pl.BlockSpec((1, RB, 16), lambda g, i: (g, i, 0)),
            pl.BlockSpec((1, RB, 16), lambda g, i: (g, i, 0)),
        ],
        out_shape=[
            jax.ShapeDtypeStruct((2, N1, D), jnp.float32),
            jax.ShapeDtypeStruct((2, N1, 16), jnp.float32),
            jax.ShapeDtypeStruct((2, N1, 16), jnp.float32),
        ],
    )(deg, deg, x)
    return h1, ns, nd


def _tc_mm_body(a_ref, w_ref, b_ref, nd_ref, ns_ref, o_ref):
    nd = nd_ref[0][:, 0:1]
    a = a_ref[0] * nd
    acc = jnp.dot(a, w_ref[0], preferred_element_type=jnp.float32)
    r = jnp.maximum(acc + b_ref[0], 0.0)
    o_ref[0] = r * ns_ref[0][:, 0:1]


def _tc_matmul_relu(agg, W, b, nd, ns):
    return pl.pallas_call(
        _tc_mm_body,
        grid=(2, NBLK),
        in_specs=[
            pl.BlockSpec((1, RB, D), lambda g, i: (g, i, 0)),
            pl.BlockSpec((1, D, D), lambda g, i: (g, 0, 0)),
            pl.BlockSpec((1, 1, D), lambda g, i: (g, 0, 0)),
            pl.BlockSpec((1, RB, 16), lambda g, i: (g, i, 0)),
            pl.BlockSpec((1, RB, 16), lambda g, i: (g, i, 0)),
        ],
        out_specs=pl.BlockSpec((1, RB, D), lambda g, i: (g, i, 0)),
        out_shape=jax.ShapeDtypeStruct((2, N1, D), jnp.float32),
    )(agg, W, b.reshape(2, 1, D), nd, ns)


def _tc_pool_body(a_ref, w_ref, b_ref, nd_ref, o_ref, acc_ref):
    g = pl.program_id(0)
    i = pl.program_id(1)
    a = a_ref[0] * nd_ref[0][:, 0:1]
    x = jnp.dot(a, w_ref[0], preferred_element_type=jnp.float32)
    x = jnp.maximum(x + b_ref[0], 0.0)
    rowid = i * RB + lax.broadcasted_iota(jnp.int32, (RB, 1), 0)
    x = jnp.where(rowid < N, x, 0.0)
    ssum = jnp.sum(x, axis=0, keepdims=True)
    smax = jnp.max(x, axis=0, keepdims=True)
    red = jnp.where(g == 0, ssum, smax)

    @pl.when(i == 0)
    def _():
        acc_ref[...] = red

    @pl.when(i > 0)
    def _():
        a0 = acc_ref[...]
        acc_ref[...] = jnp.where(g == 0, a0 + red, jnp.maximum(a0, red))

    @pl.when(i == NBLK - 1)
    def _():
        r = acc_ref[...]
        o_ref[0] = jnp.where(g == 0, r * jnp.float32(1.0 / N), r)


def _tc_matmul_pool(agg, W, b, nd):
    return pl.pallas_call(
        _tc_pool_body,
        grid=(2, NBLK),
        in_specs=[
            pl.BlockSpec((1, RB, D), lambda g, i: (g, i, 0)),
            pl.BlockSpec((1, D, D), lambda g, i: (g, 0, 0)),
            pl.BlockSpec((1, 1, D), lambda g, i: (g, 0, 0)),
            pl.BlockSpec((1, RB, 16), lambda g, i: (g, i, 0)),
        ],
        out_specs=pl.BlockSpec((1, 1, D), lambda g, i: (g, 0, 0)),
        out_shape=jax.ShapeDtypeStruct((2, 1, D), jnp.float32),
        scratch_shapes=[pltpu.VMEM((1, D), jnp.float32)],
    )(agg, W, b.reshape(2, 1, D), nd)


def _tc_head_body(p_ref, w_ref, b_ref, o_ref):
    p = p_ref[...]
    mn = jnp.min(p, axis=1, keepdims=True)
    mx = jnp.max(p, axis=1, keepdims=True)
    q = (p - mn) / (mx - mn)
    emb = q[0:1, :] + q[1:2, :]
    o_ref[...] = (
        jnp.dot(emb, w_ref[...], preferred_element_type=jnp.float32) + b_ref[...]
    )


def _tc_head(pooled, cls_W, cls_b):
    return pl.pallas_call(
        _tc_head_body,
        out_shape=jax.ShapeDtypeStruct((1, C), jnp.float32),
    )(pooled.reshape(2, D), cls_W, cls_b.reshape(1, C))


def kernel(apig_edge_index, apig_feat, fcg_edge_index, fcg_feat,
           W_a1, b_a1, W_a2, b_a2, W_f1, b_f1, W_f2, b_f2,
           attn_W, attn_b, cls_W, cls_b):
    f32 = jnp.float32
    xpad = jnp.zeros((N1 - N, D), f32)
    x = jnp.stack([jnp.concatenate([apig_feat.astype(f32), xpad], axis=0),
                   jnp.concatenate([fcg_feat.astype(f32), xpad], axis=0)])

    epad = (jnp.arange(E_PAD - E, dtype=jnp.int32) % (N1 - N)) + N

    def prep_edges(ei, g):
        srcp = jnp.concatenate([ei[0].astype(jnp.int32), epad])
        dstp = jnp.concatenate([ei[1].astype(jnp.int32), epad])
        return jnp.concatenate([srcp, dstp, srcp + jnp.int32(g * N1)])

    e2 = jnp.concatenate(
        [prep_edges(apig_edge_index, 0), prep_edges(fcg_edge_index, 1)]
    ).reshape(6 * ECHUNKS, K)

    onesD = jnp.ones((K, D), f32)
    zerosD = jnp.zeros((K, D), f32)
    sidx = (jnp.arange(K, dtype=jnp.int32) % (N1 - N)) + N

    deg = _sc_degrees(e2, onesD, zerosD)
    h1, ns, nd = _tc_norms_and_h1(deg.reshape(4, N1, D), x)

    agg1 = _sc_conv(e2, h1.reshape(2 * N1, D), zerosD, sidx)

    W1 = jnp.stack([W_a1, W_f1])
    b1 = jnp.stack([b_a1, b_f1])
    h2 = _tc_matmul_relu(agg1.reshape(2, N1, D), W1, b1, nd, ns)

    agg2 = _sc_conv(e2, h2.reshape(2 * N1, D), zerosD, sidx)

    W2 = jnp.stack([W_a2, W_f2])
    b2 = jnp.stack([b_a2, b_f2])
    pooled = _tc_matmul_pool(agg2.reshape(2, N1, D), W2, b2, nd)

    out = _tc_head(pooled, cls_W, cls_b)
    return out.reshape(C)


# speedup vs baseline: 5.9567x; 1.0968x over previous
import functools

import jax
import jax.numpy as jnp
from jax import lax
from jax.experimental import pallas as pl
from jax.experimental.pallas import tpu as pltpu
from jax.experimental.pallas import tpu_sc as plsc

N = 10000
E = 320000
D = 128
C = 10

NC = 2
NS = 16
RPT = 640
N1 = NS * RPT
K = 128
NCHG = E // K
NCHS = 2504
NBI = 8


def _tile_sched(s):
    bc = jnp.where(s < 8, 160 * s,
                   jnp.where(s < 15, 1280 + 152 * (s - 8), 2344))
    nblk = jnp.where(s < 8, 20, 19)
    return pl.multiple_of(bc, 8), nblk
RB = 2048
NBLK = N1 // RB

_MESH = plsc.VectorSubcoreMesh(core_axis_name="c", subcore_axis_name="s")


@functools.partial(
    pl.kernel,
    out_type=jax.ShapeDtypeStruct((4 * N1, D), jnp.float32),
    mesh=_MESH,
    scratch_types=[
        pltpu.VMEM_SHARED((N1, D), jnp.float32),
        pltpu.VMEM((K, D), jnp.float32),
        pltpu.VMEM((NBI, K), jnp.int32),
        pltpu.SemaphoreType.DMA,
    ],
)
def _sc_degrees(e4_ref, ones_ref, zeros_ref, deg_ref, hist, rows, idxb, sem):
    c = lax.axis_index("c")
    s = lax.axis_index("s")
    base = s * RPT
    bc, nblk = _tile_sched(s)
    tail = s == 15

    for dirn in (0, 1):
        pltpu.sync_copy(zeros_ref, rows)
        for k in range(RPT // K):
            pltpu.sync_copy(rows, hist.at[pl.ds(base + k * K, K)])
        plsc.subcore_barrier()
        pltpu.sync_copy(ones_ref, rows)
        row0 = (c * 2 + dirn) * NCHS + bc

        def blk_body(b, _):
            pltpu.sync_copy(e4_ref.at[pl.ds(row0 + b * NBI, NBI)], idxb)
            for j in range(NBI):
                pltpu.async_copy(rows, hist.at[idxb.at[j]], sem, add=True)
            for j in range(NBI):
                pltpu.make_async_copy(rows, hist.at[idxb.at[j]], sem).wait()
            return 0

        lax.fori_loop(0, nblk, blk_body, 0)

        @pl.when(tail)
        def _():
            tr = row0 + 19 * NBI
            pltpu.sync_copy(e4_ref.at[pl.ds(tr, NBI)], idxb)
            for j in range(4):
                pltpu.async_copy(rows, hist.at[idxb.at[j]], sem, add=True)
            for j in range(4):
                pltpu.make_async_copy(rows, hist.at[idxb.at[j]], sem).wait()

        plsc.subcore_barrier()

        for k in range(RPT // K):
            r0 = base + k * K
            pltpu.sync_copy(hist.at[pl.ds(r0, K)], rows)
            pltpu.sync_copy(rows, deg_ref.at[pl.ds((c * 2 + dirn) * N1 + r0, K)])


@functools.partial(
    pl.kernel,
    out_type=jax.ShapeDtypeStruct((2 * N1, D), jnp.float32),
    mesh=_MESH,
    scratch_types=[
        pltpu.VMEM_SHARED((N1, D), jnp.float32),
        pltpu.VMEM((K, D), jnp.float32),
        pltpu.VMEM((K, D), jnp.float32),
        pltpu.VMEM((NBI, K), jnp.int32),
        pltpu.VMEM((NBI, K), jnp.int32),
        pltpu.VMEM((NBI, K), jnp.int32),
        pltpu.VMEM((NBI, K), jnp.int32),
        pltpu.VMEM((K,), jnp.int32),
        pltpu.SemaphoreType.DMA,
        pltpu.SemaphoreType.DMA,
        pltpu.SemaphoreType.DMA,
        pltpu.SemaphoreType.DMA,
    ],
)
def _sc_conv(e4_ref, h_ref, zeros_ref, sidx_ref, agg_ref,
             acc, rows0, rows1, isb0, idb0, isb1, idb1, sidx,
             gs0, gs1, ss0, ss1):
    c = lax.axis_index("c")
    s = lax.axis_index("s")
    base = s * RPT
    bc, nblk = _tile_sched(s)
    row_s = (c * 2) * NCHS + bc
    row_d = (c * 2 + 1) * NCHS + bc
    coff = c * N1

    pltpu.sync_copy(zeros_ref, rows0)
    pltpu.sync_copy(zeros_ref, rows1)
    for k in range(RPT // K):
        pltpu.async_copy(rows0, acc.at[pl.ds(base + k * K, K)], gs0)
    pltpu.sync_copy(sidx_ref, sidx)
    for k in range(RPT // K):
        pltpu.make_async_copy(rows0, acc.at[pl.ds(base + k * K, K)], gs0).wait()
    plsc.subcore_barrier()

    rows_l = (rows0, rows1)
    gs_l = (gs0, gs1)
    ss_l = (ss0, ss1)
    pltpu.async_copy(rows0, acc.at[sidx], ss0, add=True)
    pltpu.async_copy(rows1, acc.at[sidx], ss1, add=True)

    def load_blk(b, isb, idb):
        pltpu.sync_copy(e4_ref.at[pl.ds(row_s + b * NBI, NBI)], isb)
        pltpu.sync_copy(e4_ref.at[pl.ds(row_d + b * NBI, NBI)], idb)
        for r in range(NBI):
            for v in range(K // 16):
                isb[r, pl.ds(v * 16, 16)] = isb[r, pl.ds(v * 16, 16)] + coff

    def do_chunk(isb, idb, j, p):
        pltpu.make_async_copy(rows_l[p], acc.at[sidx], ss_l[p]).wait()
        pltpu.async_copy(h_ref.at[isb.at[j]], rows_l[p], gs_l[p])
        pltpu.make_async_copy(h_ref.at[isb.at[j]], rows_l[p], gs_l[p]).wait()
        pltpu.async_copy(rows_l[p], acc.at[idb.at[j]], ss_l[p], add=True)

    idx_l = ((isb0, idb0), (isb1, idb1))

    def blk_pair(t, _):
        for sb in range(2):
            b = t * 2 + sb
            isb, idb = idx_l[sb]
            load_blk(b, isb, idb)
            for j in range(NBI):
                do_chunk(isb, idb, j, j & 1)
        return 0

    lax.fori_loop(0, nblk // 2, blk_pair, 0)

    @pl.when(nblk == 19)
    def _():
        load_blk(18, isb0, idb0)
        for j in range(NBI):
            do_chunk(isb0, idb0, j, j & 1)

    @pl.when(s == 15)
    def _():
        load_blk(19, isb1, idb1)
        for j in range(4):
            do_chunk(isb1, idb1, j, j & 1)

    pltpu.make_async_copy(rows0, acc.at[sidx], ss0).wait()
    pltpu.make_async_copy(rows1, acc.at[sidx], ss1).wait()
    plsc.subcore_barrier()

    for k in range(RPT // K):
        p = k & 1
        r0 = base + k * K
        if k >= 2:
            pltpu.make_async_copy(
                rows_l[p], agg_ref.at[pl.ds(c * N1 + base + (k - 2) * K, K)],
                ss_l[p]).wait()
        pltpu.async_copy(acc.at[pl.ds(r0, K)], rows_l[p], gs_l[p])
        pltpu.make_async_copy(acc.at[pl.ds(r0, K)], rows_l[p], gs_l[p]).wait()
        pltpu.async_copy(rows_l[p], agg_ref.at[pl.ds(c * N1 + r0, K)], ss_l[p])
    for k in (RPT // K - 2, RPT // K - 1):
        p = k & 1
        pltpu.make_async_copy(
            rows_l[p], agg_ref.at[pl.ds(c * N1 + base + k * K, K)],
            ss_l[p]).wait()


def _tc_norm_body(deg_s_ref, deg_d_ref, x_ref, h_ref, ns_ref, nd_ref):
    ns = lax.rsqrt(jnp.maximum(deg_s_ref[0][:, 0:1], 1.0))
    nd = lax.rsqrt(jnp.maximum(deg_d_ref[0][:, 0:1], 1.0))
    ns_ref[0] = jnp.broadcast_to(ns, (RB, 16))
    nd_ref[0] = jnp.broadcast_to(nd, (RB, 16))
    h_ref[0] = x_ref[0] * ns


def _tc_norms_and_h1(deg, x):
    h1, ns, nd = pl.pallas_call(
        _tc_norm_body,
        grid=(2, NBLK),
        in_specs=[
            pl.BlockSpec((1, RB, D), lambda g, i: (2 * g, i, 0)),
            pl.BlockSpec((1, RB, D), lambda g, i: (2 * g + 1, i, 0)),
            pl.BlockSpec((1, RB, D), lambda g, i: (g, i, 0)),
        ],
        out_specs=[
            pl.BlockSpec((1, RB, D), lambda g, i: (g, i, 0)),
            pl.BlockSpec((1, RB, 16), lambda g, i: (g, i, 0)),
            pl.BlockSpec((1, RB, 16), lambda g, i: (g, i, 0)),
        ],
        out_shape=[
            jax.ShapeDtypeStruct((2, N1, D), jnp.float32),
            jax.ShapeDtypeStruct((2, N1, 16), jnp.float32),
            jax.ShapeDtypeStruct((2, N1, 16), jnp.float32),
        ],
    )(deg, deg, x)
    return h1, ns, nd


def _tc_mm_body(a_ref, w_ref, b_ref, nd_ref, ns_ref, o_ref):
    nd = nd_ref[0][:, 0:1]
    a = a_ref[0] * nd
    acc = jnp.dot(a, w_ref[0], preferred_element_type=jnp.float32)
    r = jnp.maximum(acc + b_ref[0], 0.0)
    o_ref[0] = r * ns_ref[0][:, 0:1]


def _tc_matmul_relu(agg, W, b, nd, ns):
    return pl.pallas_call(
        _tc_mm_body,
        grid=(2, NBLK),
        in_specs=[
            pl.BlockSpec((1, RB, D), lambda g, i: (g, i, 0)),
            pl.BlockSpec((1, D, D), lambda g, i: (g, 0, 0)),
            pl.BlockSpec((1, 1, D), lambda g, i: (g, 0, 0)),
            pl.BlockSpec((1, RB, 16), lambda g, i: (g, i, 0)),
            pl.BlockSpec((1, RB, 16), lambda g, i: (g, i, 0)),
        ],
        out_specs=pl.BlockSpec((1, RB, D), lambda g, i: (g, i, 0)),
        out_shape=jax.ShapeDtypeStruct((2, N1, D), jnp.float32),
    )(agg, W, b.reshape(2, 1, D), nd, ns)


def _tc_pool_body(a_ref, w_ref, b_ref, nd_ref, o_ref, acc_ref):
    g = pl.program_id(0)
    i = pl.program_id(1)
    a = a_ref[0] * nd_ref[0][:, 0:1]
    x = jnp.dot(a, w_ref[0], preferred_element_type=jnp.float32)
    x = jnp.maximum(x + b_ref[0], 0.0)
    rowid = i * RB + lax.broadcasted_iota(jnp.int32, (RB, 1), 0)
    x = jnp.where(rowid < N, x, 0.0)
    ssum = jnp.sum(x, axis=0, keepdims=True)
    smax = jnp.max(x, axis=0, keepdims=True)
    red = jnp.where(g == 0, ssum, smax)

    @pl.when(i == 0)
    def _():
        acc_ref[...] = red

    @pl.when(i > 0)
    def _():
        a0 = acc_ref[...]
        acc_ref[...] = jnp.where(g == 0, a0 + red, jnp.maximum(a0, red))

    @pl.when(i == NBLK - 1)
    def _():
        r = acc_ref[...]
        o_ref[0] = jnp.where(g == 0, r * jnp.float32(1.0 / N), r)


def _tc_matmul_pool(agg, W, b, nd):
    return pl.pallas_call(
        _tc_pool_body,
        grid=(2, NBLK),
        in_specs=[
            pl.BlockSpec((1, RB, D), lambda g, i: (g, i, 0)),
            pl.BlockSpec((1, D, D), lambda g, i: (g, 0, 0)),
            pl.BlockSpec((1, 1, D), lambda g, i: (g, 0, 0)),
            pl.BlockSpec((1, RB, 16), lambda g, i: (g, i, 0)),
        ],
        out_specs=pl.BlockSpec((1, 1, D), lambda g, i: (g, 0, 0)),
        out_shape=jax.ShapeDtypeStruct((2, 1, D), jnp.float32),
        scratch_shapes=[pltpu.VMEM((1, D), jnp.float32)],
    )(agg, W, b.reshape(2, 1, D), nd)


def _tc_head_body(p_ref, w_ref, b_ref, o_ref):
    p = p_ref[...]
    mn = jnp.min(p, axis=1, keepdims=True)
    mx = jnp.max(p, axis=1, keepdims=True)
    q = (p - mn) / (mx - mn)
    emb = q[0:1, :] + q[1:2, :]
    o_ref[...] = (
        jnp.dot(emb, w_ref[...], preferred_element_type=jnp.float32) + b_ref[...]
    )


def _tc_head(pooled, cls_W, cls_b):
    return pl.pallas_call(
        _tc_head_body,
        out_shape=jax.ShapeDtypeStruct((1, C), jnp.float32),
    )(pooled.reshape(2, D), cls_W, cls_b.reshape(1, C))


def kernel(apig_edge_index, apig_feat, fcg_edge_index, fcg_feat,
           W_a1, b_a1, W_a2, b_a2, W_f1, b_f1, W_f2, b_f2,
           attn_W, attn_b, cls_W, cls_b):
    f32 = jnp.float32
    xpad = jnp.zeros((N1 - N, D), f32)
    x = jnp.stack([jnp.concatenate([apig_feat.astype(f32), xpad], axis=0),
                   jnp.concatenate([fcg_feat.astype(f32), xpad], axis=0)])

    er = jnp.stack([apig_edge_index.astype(jnp.int32),
                    fcg_edge_index.astype(jnp.int32)]).reshape(4, NCHG, K)
    e4 = jnp.concatenate(
        [er, jnp.zeros((4, NCHS - NCHG, K), jnp.int32)], axis=1
    ).reshape(4 * NCHS, K)

    onesD = jnp.ones((K, D), f32)
    zerosD = jnp.zeros((K, D), f32)
    sidx = (jnp.arange(K, dtype=jnp.int32) % (N1 - N)) + N

    deg = _sc_degrees(e4, onesD, zerosD)
    h1, ns, nd = _tc_norms_and_h1(deg.reshape(4, N1, D), x)

    agg1 = _sc_conv(e4, h1.reshape(2 * N1, D), zerosD, sidx)

    W1 = jnp.stack([W_a1, W_f1])
    b1 = jnp.stack([b_a1, b_f1])
    h2 = _tc_matmul_relu(agg1.reshape(2, N1, D), W1, b1, nd, ns)

    agg2 = _sc_conv(e4, h2.reshape(2 * N1, D), zerosD, sidx)

    W2 = jnp.stack([W_a2, W_f2])
    b2 = jnp.stack([b_a2, b_f2])
    pooled = _tc_matmul_pool(agg2.reshape(2, N1, D), W2, b2, nd)

    out = _tc_head(pooled, cls_W, cls_b)
    return out.reshape(C)


# speedup vs baseline: 5.9902x; 1.0056x over previous
import functools

import jax
import jax.numpy as jnp
from jax import lax
from jax.experimental import pallas as pl
from jax.experimental.pallas import tpu as pltpu
from jax.experimental.pallas import tpu_sc as plsc

N = 10000
E = 320000
D = 128
C = 10

NC = 2
NS = 16
RPT = 640
N1 = NS * RPT
K = 128
NCHG = E // K
NCHS = 2504
NBI = 8


def _tile_sched(s):
    bc = jnp.where(s < 8, 160 * s,
                   jnp.where(s < 15, 1280 + 152 * (s - 8), 2344))
    nblk = jnp.where(s < 8, 20, 19)
    return pl.multiple_of(bc, 8), nblk
RB = 2048
NBLK = N1 // RB

_MESH = plsc.VectorSubcoreMesh(core_axis_name="c", subcore_axis_name="s")


@functools.partial(
    pl.kernel,
    out_type=jax.ShapeDtypeStruct((4 * N1, D), jnp.float32),
    mesh=_MESH,
    scratch_types=[
        pltpu.VMEM_SHARED((N1, D), jnp.float32),
        pltpu.VMEM((K, D), jnp.float32),
        pltpu.VMEM((K, D), jnp.float32),
        pltpu.VMEM((NBI, K), jnp.int32),
        pltpu.SemaphoreType.DMA,
        pltpu.SemaphoreType.DMA,
        pltpu.SemaphoreType.DMA,
    ],
)
def _sc_degrees(e4_ref, ones_ref, zeros_ref, deg_ref, hist, rows, rows2, idxb,
                sem, sm0, sm1):
    c = lax.axis_index("c")
    s = lax.axis_index("s")
    base = s * RPT
    bc, nblk = _tile_sched(s)
    tail = s == 15

    rows_l = (rows, rows2)
    sm_l = (sm0, sm1)
    for dirn in (0, 1):
        pltpu.sync_copy(zeros_ref, rows)
        for k in range(RPT // K):
            pltpu.async_copy(rows, hist.at[pl.ds(base + k * K, K)], sm0)
        for k in range(RPT // K):
            pltpu.make_async_copy(
                rows, hist.at[pl.ds(base + k * K, K)], sm0).wait()
        plsc.subcore_barrier()
        pltpu.sync_copy(ones_ref, rows)
        row0 = (c * 2 + dirn) * NCHS + bc

        def blk_body(b, _):
            pltpu.sync_copy(e4_ref.at[pl.ds(row0 + b * NBI, NBI)], idxb)
            for j in range(NBI):
                pltpu.async_copy(rows, hist.at[idxb.at[j]], sem, add=True)
            for j in range(NBI):
                pltpu.make_async_copy(rows, hist.at[idxb.at[j]], sem).wait()
            return 0

        lax.fori_loop(0, nblk, blk_body, 0)

        @pl.when(tail)
        def _():
            tr = row0 + 19 * NBI
            pltpu.sync_copy(e4_ref.at[pl.ds(tr, NBI)], idxb)
            for j in range(4):
                pltpu.async_copy(rows, hist.at[idxb.at[j]], sem, add=True)
            for j in range(4):
                pltpu.make_async_copy(rows, hist.at[idxb.at[j]], sem).wait()

        plsc.subcore_barrier()

        o0 = (c * 2 + dirn) * N1 + base
        for k in range(RPT // K):
            p = k & 1
            r0 = base + k * K
            if k >= 2:
                pltpu.make_async_copy(
                    rows_l[p], deg_ref.at[pl.ds(o0 + (k - 2) * K, K)],
                    sm_l[p]).wait()
            pltpu.async_copy(hist.at[pl.ds(r0, K)], rows_l[p], sem)
            pltpu.make_async_copy(hist.at[pl.ds(r0, K)], rows_l[p], sem).wait()
            pltpu.async_copy(rows_l[p], deg_ref.at[pl.ds(o0 + k * K, K)],
                             sm_l[p])
        for k in (RPT // K - 2, RPT // K - 1):
            p = k & 1
            pltpu.make_async_copy(
                rows_l[p], deg_ref.at[pl.ds(o0 + k * K, K)], sm_l[p]).wait()


@functools.partial(
    pl.kernel,
    out_type=jax.ShapeDtypeStruct((2 * N1, D), jnp.float32),
    mesh=_MESH,
    scratch_types=[
        pltpu.VMEM_SHARED((N1, D), jnp.float32),
        pltpu.VMEM((K, D), jnp.float32),
        pltpu.VMEM((K, D), jnp.float32),
        pltpu.VMEM((NBI, K), jnp.int32),
        pltpu.VMEM((NBI, K), jnp.int32),
        pltpu.VMEM((NBI, K), jnp.int32),
        pltpu.VMEM((NBI, K), jnp.int32),
        pltpu.VMEM((K,), jnp.int32),
        pltpu.SemaphoreType.DMA,
        pltpu.SemaphoreType.DMA,
        pltpu.SemaphoreType.DMA,
        pltpu.SemaphoreType.DMA,
    ],
)
def _sc_conv(e4_ref, h_ref, zeros_ref, sidx_ref, agg_ref,
             acc, rows0, rows1, isb0, idb0, isb1, idb1, sidx,
             gs0, gs1, ss0, ss1):
    c = lax.axis_index("c")
    s = lax.axis_index("s")
    base = s * RPT
    bc, nblk = _tile_sched(s)
    row_s = (c * 2) * NCHS + bc
    row_d = (c * 2 + 1) * NCHS + bc
    coff = c * N1

    pltpu.sync_copy(zeros_ref, rows0)
    pltpu.sync_copy(zeros_ref, rows1)
    for k in range(RPT // K):
        pltpu.async_copy(rows0, acc.at[pl.ds(base + k * K, K)], gs0)
    pltpu.sync_copy(sidx_ref, sidx)
    for k in range(RPT // K):
        pltpu.make_async_copy(rows0, acc.at[pl.ds(base + k * K, K)], gs0).wait()
    plsc.subcore_barrier()

    rows_l = (rows0, rows1)
    gs_l = (gs0, gs1)
    ss_l = (ss0, ss1)
    pltpu.async_copy(rows0, acc.at[sidx], ss0, add=True)
    pltpu.async_copy(rows1, acc.at[sidx], ss1, add=True)

    def load_blk(b, isb, idb):
        pltpu.sync_copy(e4_ref.at[pl.ds(row_s + b * NBI, NBI)], isb)
        pltpu.sync_copy(e4_ref.at[pl.ds(row_d + b * NBI, NBI)], idb)
        for r in range(NBI):
            for v in range(K // 16):
                isb[r, pl.ds(v * 16, 16)] = isb[r, pl.ds(v * 16, 16)] + coff

    def do_chunk(isb, idb, j, p):
        pltpu.make_async_copy(rows_l[p], acc.at[sidx], ss_l[p]).wait()
        pltpu.async_copy(h_ref.at[isb.at[j]], rows_l[p], gs_l[p])
        pltpu.make_async_copy(h_ref.at[isb.at[j]], rows_l[p], gs_l[p]).wait()
        pltpu.async_copy(rows_l[p], acc.at[idb.at[j]], ss_l[p], add=True)

    idx_l = ((isb0, idb0), (isb1, idb1))

    def blk_pair(t, _):
        for sb in range(2):
            b = t * 2 + sb
            isb, idb = idx_l[sb]
            load_blk(b, isb, idb)
            for j in range(NBI):
                do_chunk(isb, idb, j, j & 1)
        return 0

    lax.fori_loop(0, nblk // 2, blk_pair, 0)

    @pl.when(nblk == 19)
    def _():
        load_blk(18, isb0, idb0)
        for j in range(NBI):
            do_chunk(isb0, idb0, j, j & 1)

    @pl.when(s == 15)
    def _():
        load_blk(19, isb1, idb1)
        for j in range(4):
            do_chunk(isb1, idb1, j, j & 1)

    pltpu.make_async_copy(rows0, acc.at[sidx], ss0).wait()
    pltpu.make_async_copy(rows1, acc.at[sidx], ss1).wait()
    plsc.subcore_barrier()

    for k in range(RPT // K):
        p = k & 1
        r0 = base + k * K
        if k >= 2:
            pltpu.make_async_copy(
                rows_l[p], agg_ref.at[pl.ds(c * N1 + base + (k - 2) * K, K)],
                ss_l[p]).wait()
        pltpu.async_copy(acc.at[pl.ds(r0, K)], rows_l[p], gs_l[p])
        pltpu.make_async_copy(acc.at[pl.ds(r0, K)], rows_l[p], gs_l[p]).wait()
        pltpu.async_copy(rows_l[p], agg_ref.at[pl.ds(c * N1 + r0, K)], ss_l[p])
    for k in (RPT // K - 2, RPT // K - 1):
        p = k & 1
        pltpu.make_async_copy(
            rows_l[p], agg_ref.at[pl.ds(c * N1 + base + k * K, K)],
            ss_l[p]).wait()


def _tc_norm_body(deg_s_ref, deg_d_ref, x_ref, h_ref, ns_ref, nd_ref):
    ns = lax.rsqrt(jnp.maximum(deg_s_ref[0][:, 0:1], 1.0))
    nd = lax.rsqrt(jnp.maximum(deg_d_ref[0][:, 0:1], 1.0))
    ns_ref[0] = jnp.broadcast_to(ns, (RB, 16))
    nd_ref[0] = jnp.broadcast_to(nd, (RB, 16))
    h_ref[0] = x_ref[0] * ns


def _tc_norms_and_h1(deg, x):
    h1, ns, nd = pl.pallas_call(
        _tc_norm_body,
        grid=(2, NBLK),
        in_specs=[
            pl.BlockSpec((1, RB, D), lambda g, i: (2 * g, i, 0)),
            pl.BlockSpec((1, RB, D), lambda g, i: (2 * g + 1, i, 0)),
            pl.BlockSpec((1, RB, D), lambda g, i: (g, i, 0)),
        ],
        out_specs=[
            pl.BlockSpec((1, RB, D), lambda g, i: (g, i, 0)),
            pl.BlockSpec((1, RB, 16), lambda g, i: (g, i, 0)),
            pl.BlockSpec((1, RB, 16), lambda g, i: (g, i, 0)),
        ],
        out_shape=[
            jax.ShapeDtypeStruct((2, N1, D), jnp.float32),
            jax.ShapeDtypeStruct((2, N1, 16), jnp.float32),
            jax.ShapeDtypeStruct((2, N1, 16), jnp.float32),
        ],
    )(deg, deg, x)
    return h1, ns, nd


def _tc_mm_body(a_ref, w_ref, b_ref, nd_ref, ns_ref, o_ref):
    nd = nd_ref[0][:, 0:1]
    a = a_ref[0] * nd
    acc = jnp.dot(a, w_ref[0], preferred_element_type=jnp.float32)
    r = jnp.maximum(acc + b_ref[0], 0.0)
    o_ref[0] = r * ns_ref[0][:, 0:1]


def _tc_matmul_relu(agg, W, b, nd, ns):
    return pl.pallas_call(
        _tc_mm_body,
        grid=(2, NBLK),
        in_specs=[
            pl.BlockSpec((1, RB, D), lambda g, i: (g, i, 0)),
            pl.BlockSpec((1, D, D), lambda g, i: (g, 0, 0)),
            pl.BlockSpec((1, 1, D), lambda g, i: (g, 0, 0)),
            pl.BlockSpec((1, RB, 16), lambda g, i: (g, i, 0)),
            pl.BlockSpec((1, RB, 16), lambda g, i: (g, i, 0)),
        ],
        out_specs=pl.BlockSpec((1, RB, D), lambda g, i: (g, i, 0)),
        out_shape=jax.ShapeDtypeStruct((2, N1, D), jnp.float32),
    )(agg, W, b.reshape(2, 1, D), nd, ns)


def _tc_pool_body(a_ref, w_ref, b_ref, nd_ref, o_ref, acc_ref):
    g = pl.program_id(0)
    i = pl.program_id(1)
    a = a_ref[0] * nd_ref[0][:, 0:1]
    x = jnp.dot(a, w_ref[0], preferred_element_type=jnp.float32)
    x = jnp.maximum(x + b_ref[0], 0.0)
    rowid = i * RB + lax.broadcasted_iota(jnp.int32, (RB, 1), 0)
    x = jnp.where(rowid < N, x, 0.0)
    ssum = jnp.sum(x, axis=0, keepdims=True)
    smax = jnp.max(x, axis=0, keepdims=True)
    red = jnp.where(g == 0, ssum, smax)

    @pl.when(i == 0)
    def _():
        acc_ref[...] = red

    @pl.when(i > 0)
    def _():
        a0 = acc_ref[...]
        acc_ref[...] = jnp.where(g == 0, a0 + red, jnp.maximum(a0, red))

    @pl.when(i == NBLK - 1)
    def _():
        r = acc_ref[...]
        o_ref[0] = jnp.where(g == 0, r * jnp.float32(1.0 / N), r)


def _tc_matmul_pool(agg, W, b, nd):
    return pl.pallas_call(
        _tc_pool_body,
        grid=(2, NBLK),
        in_specs=[
            pl.BlockSpec((1, RB, D), lambda g, i: (g, i, 0)),
            pl.BlockSpec((1, D, D), lambda g, i: (g, 0, 0)),
            pl.BlockSpec((1, 1, D), lambda g, i: (g, 0, 0)),
            pl.BlockSpec((1, RB, 16), lambda g, i: (g, i, 0)),
        ],
        out_specs=pl.BlockSpec((1, 1, D), lambda g, i: (g, 0, 0)),
        out_shape=jax.ShapeDtypeStruct((2, 1, D), jnp.float32),
        scratch_shapes=[pltpu.VMEM((1, D), jnp.float32)],
    )(agg, W, b.reshape(2, 1, D), nd)


def _tc_head_body(p_ref, w_ref, b_ref, o_ref):
    p = p_ref[...]
    mn = jnp.min(p, axis=1, keepdims=True)
    mx = jnp.max(p, axis=1, keepdims=True)
    q = (p - mn) / (mx - mn)
    emb = q[0:1, :] + q[1:2, :]
    o_ref[...] = (
        jnp.dot(emb, w_ref[...], preferred_element_type=jnp.float32) + b_ref[...]
    )


def _tc_head(pooled, cls_W, cls_b):
    return pl.pallas_call(
        _tc_head_body,
        out_shape=jax.ShapeDtypeStruct((1, C), jnp.float32),
    )(pooled.reshape(2, D), cls_W, cls_b.reshape(1, C))


def kernel(apig_edge_index, apig_feat, fcg_edge_index, fcg_feat,
           W_a1, b_a1, W_a2, b_a2, W_f1, b_f1, W_f2, b_f2,
           attn_W, attn_b, cls_W, cls_b):
    f32 = jnp.float32
    xpad = jnp.zeros((N1 - N, D), f32)
    x = jnp.stack([jnp.concatenate([apig_feat.astype(f32), xpad], axis=0),
                   jnp.concatenate([fcg_feat.astype(f32), xpad], axis=0)])

    er = jnp.stack([apig_edge_index.astype(jnp.int32),
                    fcg_edge_index.astype(jnp.int32)]).reshape(4, NCHG, K)
    e4 = jnp.concatenate(
        [er, jnp.zeros((4, NCHS - NCHG, K), jnp.int32)], axis=1
    ).reshape(4 * NCHS, K)

    onesD = jnp.ones((K, D), f32)
    zerosD = jnp.zeros((K, D), f32)
    sidx = (jnp.arange(K, dtype=jnp.int32) % (N1 - N)) + N

    deg = _sc_degrees(e4, onesD, zerosD)
    h1, ns, nd = _tc_norms_and_h1(deg.reshape(4, N1, D), x)

    agg1 = _sc_conv(e4, h1.reshape(2 * N1, D), zerosD, sidx)

    W1 = jnp.stack([W_a1, W_f1])
    b1 = jnp.stack([b_a1, b_f1])
    h2 = _tc_matmul_relu(agg1.reshape(2, N1, D), W1, b1, nd, ns)

    agg2 = _sc_conv(e4, h2.reshape(2 * N1, D), zerosD, sidx)

    W2 = jnp.stack([W_a2, W_f2])
    b2 = jnp.stack([b_a2, b_f2])
    pooled = _tc_matmul_pool(agg2.reshape(2, N1, D), W2, b2, nd)

    out = _tc_head(pooled, cls_W, cls_b)
    return out.reshape(C)
